# Initial kernel scaffold; baseline (speedup 1.0000x reference)
import jax
import jax.numpy as jnp
from jax.experimental import pallas as pl


def kernel(x):
    raise NotImplementedError("write your pallas kernel here")


# speedup vs baseline: 2.0304x; 2.0304x over previous
import functools

import jax
import jax.numpy as jnp
from jax import lax
from jax.experimental import pallas as pl
from jax.experimental.pallas import tpu as pltpu
from jax.experimental.pallas import tpu_sc as plsc

_P = 4
_NC = 2
_NS = 16
_NW = _NC * _NS
_L = 16

_CHUNK = 8192
_OUT_CHUNK = _CHUNK * _P


@functools.lru_cache(maxsize=None)
def _make_sc_call(n_total: int):
    n_w = n_total // _NW
    n_chunks = n_w // _CHUNK
    out_total = n_total * _P
    mesh = plsc.VectorSubcoreMesh(core_axis_name="c", subcore_axis_name="s")

    @functools.partial(
        pl.kernel,
        mesh=mesh,
        out_type=jax.ShapeDtypeStruct((out_total,), jnp.float32),
        scratch_types=[
            pltpu.VMEM((_CHUNK,), jnp.float32),
            pltpu.VMEM((_OUT_CHUNK,), jnp.float32),
        ],
        compiler_params=pltpu.CompilerParams(needs_layout_passes=False),
    )
    def sc_interp(x_hbm, y_hbm, in_v, out_v):
        wid = lax.axis_index("s") * _NC + lax.axis_index("c")
        zeros = jnp.zeros((_L,), jnp.float32)

        def zero_body(j, carry):
            out_v[pl.ds(j * _L, _L)] = zeros
            return carry

        lax.fori_loop(0, _OUT_CHUNK // _L, zero_body, 0)

        lane_idx = lax.iota(jnp.int32, _L) * _P
        in_base = wid * n_w
        out_base = wid * (n_w * _P)

        def chunk_body(c, carry):
            src = x_hbm.at[pl.ds(pl.multiple_of(in_base + c * _CHUNK, 8), _CHUNK)]
            pltpu.sync_copy(src, in_v)

            def scat_body(j, idx):
                v = in_v[pl.ds(j * _L, _L)]
                plsc.store_scatter(out_v, [idx], v)
                return idx + (_L * _P)

            lax.fori_loop(0, _CHUNK // _L, scat_body, lane_idx)

            dst = y_hbm.at[
                pl.ds(pl.multiple_of(out_base + c * _OUT_CHUNK, 8), _OUT_CHUNK)
            ]
            pltpu.sync_copy(out_v, dst)
            return carry

        lax.fori_loop(0, n_chunks, chunk_body, 0)

    return sc_interp


def kernel(x):
    b, t = x.shape
    n_total = b * t
    y = _make_sc_call(n_total)(x.reshape(n_total))
    return y.reshape(b, t * _P)


# speedup vs baseline: 2.7294x; 1.3442x over previous
import functools

import jax
import jax.numpy as jnp
from jax import lax
from jax.experimental import pallas as pl
from jax.experimental.pallas import tpu as pltpu
from jax.experimental.pallas import tpu_sc as plsc

_P = 4
_NC = 2
_NS = 16
_NW = _NC * _NS
_L = 16

_CHUNK = 8192
_OUT_CHUNK = _CHUNK * _P


@functools.lru_cache(maxsize=None)
def _make_sc_call(n_total: int):
    n_w = n_total // _NW
    n_chunks = n_w // _CHUNK
    assert n_chunks % 2 == 0 and n_chunks >= 6
    out_total = n_total * _P
    mesh = plsc.VectorSubcoreMesh(core_axis_name="c", subcore_axis_name="s")

    @functools.partial(
        pl.kernel,
        mesh=mesh,
        out_type=jax.ShapeDtypeStruct((out_total,), jnp.float32),
        scratch_types=[
            pltpu.VMEM((_CHUNK,), jnp.float32),
            pltpu.VMEM((_CHUNK,), jnp.float32),
            pltpu.VMEM((_OUT_CHUNK,), jnp.float32),
            pltpu.VMEM((_OUT_CHUNK,), jnp.float32),
            pltpu.SemaphoreType.DMA,
            pltpu.SemaphoreType.DMA,
            pltpu.SemaphoreType.DMA,
            pltpu.SemaphoreType.DMA,
        ],
        compiler_params=pltpu.CompilerParams(needs_layout_passes=False),
    )
    def sc_interp(x_hbm, y_hbm, in0, in1, out0, out1, si0, si1, so0, so1):
        wid = lax.axis_index("s") * _NC + lax.axis_index("c")
        in_base = wid * n_w
        out_base = wid * (n_w * _P)
        ins = (in0, in1)
        outs = (out0, out1)
        sis = (si0, si1)
        sos = (so0, so1)

        zeros = jnp.zeros((_L,), jnp.float32)

        @plsc.parallel_loop(0, _OUT_CHUNK // _L, unroll=8)
        def _(j):
            out0[pl.ds(j * _L, _L)] = zeros
            out1[pl.ds(j * _L, _L)] = zeros

        lane_idx = lax.iota(jnp.int32, _L) * _P

        def in_slice(c):
            return x_hbm.at[pl.ds(pl.multiple_of(in_base + c * _CHUNK, 8), _CHUNK)]

        def out_slice(c):
            return y_hbm.at[
                pl.ds(pl.multiple_of(out_base + c * _OUT_CHUNK, 8), _OUT_CHUNK)
            ]

        def scatter(b):
            @plsc.parallel_loop(0, _CHUNK // _L, unroll=8)
            def _(j):
                v = ins[b][pl.ds(j * _L, _L)]
                plsc.store_scatter(outs[b], [lane_idx + j * (_L * _P)], v)

        def chunk_body(c, b, prefetch, wait_out):
            if wait_out:
                pltpu.make_async_copy(outs[b], out_slice(c), sos[b]).wait()
            pltpu.make_async_copy(in_slice(c), ins[b], sis[b]).wait()
            scatter(b)
            if prefetch:
                pltpu.async_copy(in_slice(c + 2), ins[b], sis[b])
            pltpu.async_copy(outs[b], out_slice(c), sos[b])

        for b in range(2):
            pltpu.async_copy(in_slice(b), ins[b], sis[b])
        for b in range(2):
            chunk_body(b, b, prefetch=True, wait_out=False)

        def pair_body(p, carry):
            for b in range(2):
                chunk_body(2 * p + b, b, prefetch=True, wait_out=True)
            return carry

        lax.fori_loop(1, n_chunks // 2 - 1, pair_body, 0)

        for b in range(2):
            chunk_body(n_chunks - 2 + b, b, prefetch=False, wait_out=True)

        for b in range(2):
            pltpu.make_async_copy(
                outs[b], out_slice(n_chunks - 2 + b), sos[b]
            ).wait()

    return sc_interp


def kernel(x):
    b, t = x.shape
    n_total = b * t
    y = _make_sc_call(n_total)(x.reshape(n_total))
    return y.reshape(b, t * _P)


# speedup vs baseline: 7.2112x; 2.6421x over previous
import functools

import jax
import jax.numpy as jnp
from jax import lax
from jax.experimental import pallas as pl
from jax.experimental.pallas import tpu as pltpu
from jax.experimental.pallas import tpu_sc as plsc

_P = 4
_NC = 2
_NS = 16
_NW = _NC * _NS
_L = 16

_RB = 8
_W_PER_RB = 4
_CHUNK_C = 1024
_OUT_C = _CHUNK_C * _P
_VREGS = _RB * _CHUNK_C // _L


@functools.lru_cache(maxsize=None)
def _make_sc_call(n_rows: int, n_cols: int):
    assert n_rows % _RB == 0 and n_rows // _RB == _NW // _W_PER_RB
    cols_w = n_cols // _W_PER_RB
    n_chunks = cols_w // _CHUNK_C
    assert n_chunks % 2 == 0 and n_chunks >= 6
    mesh = plsc.VectorSubcoreMesh(core_axis_name="c", subcore_axis_name="s")

    @functools.partial(
        pl.kernel,
        mesh=mesh,
        out_type=jax.ShapeDtypeStruct((n_rows, n_cols * _P), jnp.float32),
        scratch_types=[
            pltpu.VMEM((_RB, _CHUNK_C), jnp.float32),
            pltpu.VMEM((_RB, _CHUNK_C), jnp.float32),
            pltpu.VMEM((_RB, _OUT_C), jnp.float32),
            pltpu.VMEM((_RB, _OUT_C), jnp.float32),
            pltpu.SemaphoreType.DMA,
            pltpu.SemaphoreType.DMA,
            pltpu.SemaphoreType.DMA,
            pltpu.SemaphoreType.DMA,
        ],
        compiler_params=pltpu.CompilerParams(needs_layout_passes=False),
    )
    def sc_interp(x_hbm, y_hbm, in0, in1, out0, out1, si0, si1, so0, so1):
        wid = lax.axis_index("s") * _NC + lax.axis_index("c")
        rb = wid // _W_PER_RB
        q = wid % _W_PER_RB
        row0 = pl.multiple_of(rb * _RB, _RB)
        col0 = q * cols_w
        ins = (in0, in1)
        outs = (out0, out1)
        sis = (si0, si1)
        sos = (so0, so1)

        zeros = jnp.zeros((_L,), jnp.float32)

        @plsc.parallel_loop(0, _RB * _OUT_C // _L, unroll=8)
        def _(j):
            s = jax.lax.shift_right_logical(j, 8)
            c = jax.lax.shift_left(j & (_OUT_C // _L - 1), 4)
            c = pl.multiple_of(c, _L)
            out0[s, pl.ds(c, _L)] = zeros
            out1[s, pl.ds(c, _L)] = zeros

        lane_idx = lax.iota(jnp.int32, _L) * _P

        def in_slice(c):
            return x_hbm.at[
                pl.ds(row0, _RB),
                pl.ds(pl.multiple_of(col0 + c * _CHUNK_C, _CHUNK_C), _CHUNK_C),
            ]

        def out_slice(c):
            return y_hbm.at[
                pl.ds(row0, _RB),
                pl.ds(pl.multiple_of((col0 + c * _CHUNK_C) * _P, _OUT_C), _OUT_C),
            ]

        def scatter(b):
            @plsc.parallel_loop(0, _VREGS, unroll=8)
            def _(j):
                s = jax.lax.shift_right_logical(j, 6)
                c = jax.lax.shift_left(j & (_CHUNK_C // _L - 1), 4)
                c = pl.multiple_of(c, _L)
                v = ins[b][s, pl.ds(c, _L)]
                rows = jnp.broadcast_to(s, (_L,))
                cols = lane_idx + c * _P
                plsc.store_scatter(outs[b], [rows, cols], v)

        def chunk_body(c, b, prefetch, wait_out):
            if wait_out:
                pltpu.make_async_copy(outs[b], out_slice(c), sos[b]).wait()
            pltpu.make_async_copy(in_slice(c), ins[b], sis[b]).wait()
            scatter(b)
            if prefetch:
                pltpu.async_copy(in_slice(c + 2), ins[b], sis[b])
            pltpu.async_copy(outs[b], out_slice(c), sos[b])

        for b in range(2):
            pltpu.async_copy(in_slice(b), ins[b], sis[b])
        for b in range(2):
            chunk_body(b, b, prefetch=True, wait_out=False)

        def pair_body(p, carry):
            for b in range(2):
                chunk_body(2 * p + b, b, prefetch=True, wait_out=True)
            return carry

        lax.fori_loop(1, n_chunks // 2 - 1, pair_body, 0)

        for b in range(2):
            chunk_body(n_chunks - 2 + b, b, prefetch=False, wait_out=True)

        for b in range(2):
            pltpu.make_async_copy(
                outs[b], out_slice(n_chunks - 2 + b), sos[b]
            ).wait()

    return sc_interp


def kernel(x):
    b, t = x.shape
    return _make_sc_call(b, t)(x)


# speedup vs baseline: 7.2445x; 1.0046x over previous
import functools

import jax
import jax.numpy as jnp
from jax import lax
from jax.experimental import pallas as pl
from jax.experimental.pallas import tpu as pltpu
from jax.experimental.pallas import tpu_sc as plsc

_P = 4
_NC = 2
_NS = 16
_NW = _NC * _NS
_L = 16

_RB = 8
_W_PER_RB = 4
_CHUNK_C = 1024
_OUT_C = _CHUNK_C * _P
_VREGS = _RB * _CHUNK_C // _L


@functools.lru_cache(maxsize=None)
def _make_sc_call(n_rows: int, n_cols: int):
    assert n_rows % _RB == 0 and n_rows // _RB == _NW // _W_PER_RB
    cols_w = n_cols // _W_PER_RB
    n_chunks = cols_w // _CHUNK_C
    assert n_chunks % 2 == 0 and n_chunks >= 6
    mesh = plsc.VectorSubcoreMesh(core_axis_name="c", subcore_axis_name="s")

    @functools.partial(
        pl.kernel,
        mesh=mesh,
        out_type=jax.ShapeDtypeStruct((n_rows, n_cols * _P), jnp.float32),
        scratch_types=[
            pltpu.VMEM((_RB, _CHUNK_C), jnp.float32),
            pltpu.VMEM((_RB, _CHUNK_C), jnp.float32),
            pltpu.VMEM((_RB, _OUT_C), jnp.float32),
            pltpu.VMEM((_RB, _OUT_C), jnp.float32),
            pltpu.SemaphoreType.DMA,
            pltpu.SemaphoreType.DMA,
            pltpu.SemaphoreType.DMA,
            pltpu.SemaphoreType.DMA,
        ],
        compiler_params=pltpu.CompilerParams(needs_layout_passes=False),
    )
    def sc_interp(x_hbm, y_hbm, in0, in1, out0, out1, si0, si1, so0, so1):
        wid = lax.axis_index("s") * _NC + lax.axis_index("c")
        rb = wid // _W_PER_RB
        q = wid % _W_PER_RB
        row0 = pl.multiple_of(rb * _RB, _RB)
        col0 = q * cols_w
        ins = (in0, in1)
        outs = (out0, out1)
        sis = (si0, si1)
        sos = (so0, so1)

        zeros = jnp.zeros((_L,), jnp.float32)

        @plsc.parallel_loop(0, _RB * _OUT_C // _L, unroll=8)
        def _(j):
            s = jax.lax.shift_right_logical(j, 8)
            c = jax.lax.shift_left(j & (_OUT_C // _L - 1), 4)
            c = pl.multiple_of(c, _L)
            out0[s, pl.ds(c, _L)] = zeros
            out1[s, pl.ds(c, _L)] = zeros

        lane_idx = lax.iota(jnp.int32, _L) * _P

        def in_slice(c):
            return x_hbm.at[
                pl.ds(row0, _RB),
                pl.ds(pl.multiple_of(col0 + c * _CHUNK_C, _CHUNK_C), _CHUNK_C),
            ]

        def out_slice(c):
            return y_hbm.at[
                pl.ds(row0, _RB),
                pl.ds(pl.multiple_of((col0 + c * _CHUNK_C) * _P, _OUT_C), _OUT_C),
            ]

        def scatter(b):
            @plsc.parallel_loop(0, _VREGS, unroll=8)
            def _(j):
                s = jax.lax.shift_right_logical(j, 6)
                c = jax.lax.shift_left(j & (_CHUNK_C // _L - 1), 4)
                c = pl.multiple_of(c, _L)
                v = ins[b][s, pl.ds(c, _L)]
                rows = jnp.broadcast_to(s, (_L,))
                cols = lane_idx + c * _P
                plsc.store_scatter(outs[b], [rows, cols], v)

        for b in range(2):
            pltpu.async_copy(in_slice(b), ins[b], sis[b])

        def pair_body(p, carry):
            for b in range(2):
                c = 2 * p + b

                @pl.when(c >= 2)
                def _():
                    pltpu.make_async_copy(outs[b], out_slice(c), sos[b]).wait()

                pltpu.make_async_copy(in_slice(c), ins[b], sis[b]).wait()
                scatter(b)

                @pl.when(c + 2 < n_chunks)
                def _():
                    pltpu.async_copy(in_slice(c + 2), ins[b], sis[b])

                pltpu.async_copy(outs[b], out_slice(c), sos[b])
            return carry

        lax.fori_loop(0, n_chunks // 2, pair_body, 0)

        for b in range(2):
            pltpu.make_async_copy(
                outs[b], out_slice(n_chunks - 2 + b), sos[b]
            ).wait()

    return sc_interp


def kernel(x):
    b, t = x.shape
    return _make_sc_call(b, t)(x)


# speedup vs baseline: 7.3017x; 1.0079x over previous
import functools

import jax
import jax.numpy as jnp
from jax import lax
from jax.experimental import pallas as pl
from jax.experimental.pallas import tpu as pltpu
from jax.experimental.pallas import tpu_sc as plsc

_P = 4
_NC = 2
_NS = 16
_NW = _NC * _NS
_L = 16

_RB = 8
_W_PER_RB = 4
_CHUNK_C = 512
_OUT_C = _CHUNK_C * _P
_VREGS = _RB * _CHUNK_C // _L


@functools.lru_cache(maxsize=None)
def _make_sc_call(n_rows: int, n_cols: int):
    assert n_rows % _RB == 0 and n_rows // _RB == _NW // _W_PER_RB
    cols_w = n_cols // _W_PER_RB
    n_chunks = cols_w // _CHUNK_C
    assert n_chunks % 2 == 0 and n_chunks >= 6
    mesh = plsc.VectorSubcoreMesh(core_axis_name="c", subcore_axis_name="s")

    @functools.partial(
        pl.kernel,
        mesh=mesh,
        out_type=jax.ShapeDtypeStruct((n_rows, n_cols * _P), jnp.float32),
        scratch_types=[
            pltpu.VMEM((_RB, _CHUNK_C), jnp.float32),
            pltpu.VMEM((_RB, _CHUNK_C), jnp.float32),
            pltpu.VMEM((_RB, _OUT_C), jnp.float32),
            pltpu.VMEM((_RB, _OUT_C), jnp.float32),
            pltpu.SemaphoreType.DMA,
            pltpu.SemaphoreType.DMA,
            pltpu.SemaphoreType.DMA,
            pltpu.SemaphoreType.DMA,
        ],
        compiler_params=pltpu.CompilerParams(needs_layout_passes=False),
    )
    def sc_interp(x_hbm, y_hbm, in0, in1, out0, out1, si0, si1, so0, so1):
        wid = lax.axis_index("s") * _NC + lax.axis_index("c")
        rb = wid // _W_PER_RB
        q = wid % _W_PER_RB
        row0 = pl.multiple_of(rb * _RB, _RB)
        col0 = q * cols_w
        ins = (in0, in1)
        outs = (out0, out1)
        sis = (si0, si1)
        sos = (so0, so1)

        zeros = jnp.zeros((_L,), jnp.float32)

        @plsc.parallel_loop(0, _RB * _OUT_C // _L, unroll=8)
        def _(j):
            s = jax.lax.shift_right_logical(j, 8)
            c = jax.lax.shift_left(j & (_OUT_C // _L - 1), 4)
            c = pl.multiple_of(c, _L)
            out0[s, pl.ds(c, _L)] = zeros
            out1[s, pl.ds(c, _L)] = zeros

        lane_idx = lax.iota(jnp.int32, _L) * _P

        def in_slice(c):
            return x_hbm.at[
                pl.ds(row0, _RB),
                pl.ds(pl.multiple_of(col0 + c * _CHUNK_C, _CHUNK_C), _CHUNK_C),
            ]

        def out_slice(c):
            return y_hbm.at[
                pl.ds(row0, _RB),
                pl.ds(pl.multiple_of((col0 + c * _CHUNK_C) * _P, _OUT_C), _OUT_C),
            ]

        def scatter(b):
            @plsc.parallel_loop(0, _VREGS, unroll=8)
            def _(j):
                s = jax.lax.shift_right_logical(j, 6)
                c = jax.lax.shift_left(j & (_CHUNK_C // _L - 1), 4)
                c = pl.multiple_of(c, _L)
                v = ins[b][s, pl.ds(c, _L)]
                rows = jnp.broadcast_to(s, (_L,))
                cols = lane_idx + c * _P
                plsc.store_scatter(outs[b], [rows, cols], v)

        for b in range(2):
            pltpu.async_copy(in_slice(b), ins[b], sis[b])

        def pair_body(p, carry):
            for b in range(2):
                c = 2 * p + b

                @pl.when(c >= 2)
                def _():
                    pltpu.make_async_copy(outs[b], out_slice(c), sos[b]).wait()

                pltpu.make_async_copy(in_slice(c), ins[b], sis[b]).wait()
                scatter(b)

                @pl.when(c + 2 < n_chunks)
                def _():
                    pltpu.async_copy(in_slice(c + 2), ins[b], sis[b])

                pltpu.async_copy(outs[b], out_slice(c), sos[b])
            return carry

        lax.fori_loop(0, n_chunks // 2, pair_body, 0)

        for b in range(2):
            pltpu.make_async_copy(
                outs[b], out_slice(n_chunks - 2 + b), sos[b]
            ).wait()

    return sc_interp


def kernel(x):
    b, t = x.shape
    return _make_sc_call(b, t)(x)
